# baseline (speedup 1.0000x reference)
import functools

import jax
import jax.numpy as jnp
from jax.experimental import pallas as pl
from jax.experimental.pallas import tpu as pltpu

GRID = 24
RES = 0.25
HALF = GRID * RES / 2.0
NCELL = GRID * GRID
NPAD = 640
CHUNK = 4096
F = 128
C1 = 64
C2 = 32
POOL = 4
PG = GRID // POOL
EMB = C2 * PG * PG

_ENC_W = 0
_ENC_B = 1
_W1 = 2
_B1 = _W1 + 9 * C1
_W2 = _B1 + 1
_B2 = _W2 + 9 * C2
_HEADS = _B2 + 1
_HSZ = EMB + 1 + 128 + 1 + 32 + 1
_ROWS = _HEADS + 4 * _HSZ
_HOUT = (8, 6, 1, 1)


def _fused_kernel(x_ref, y_ref, v_ref, wp_ref, clf_ref, reg_ref, cyc_ref,
                  acc_ref, *, num_chunks):
    i = pl.program_id(0)

    @pl.when(i == 0)
    def _init():
        acc_ref[...] = jnp.zeros_like(acc_ref)

    x = x_ref[0]
    y = y_ref[0]
    v = v_ref[0]
    cx = ((x + HALF) / RES).astype(jnp.int32)
    cy = ((y + HALF) / RES).astype(jnp.int32)
    mask = (cx >= 0) & (cx < GRID) & (cy >= 0) & (cy < GRID)
    idx = jnp.where(mask, cx * GRID + cy, NCELL)

    encw = jnp.transpose(wp_ref[_ENC_W:_ENC_W + 1, :])
    encb = jnp.transpose(wp_ref[_ENC_B:_ENC_B + 1, :])
    featT = jax.nn.relu(encw * v + encb)

    cell_ids = jax.lax.broadcasted_iota(jnp.int32, (NPAD, CHUNK), 0)
    onehotT = (cell_ids == idx).astype(jnp.float32)

    acc_ref[...] += jax.lax.dot_general(
        featT, onehotT, (((1,), (1,)), ((), ())),
        preferred_element_type=jnp.float32)

    @pl.when(i == num_chunks - 1)
    def _tail():
        gt = acc_ref[:, :NCELL]

        r = jax.lax.broadcasted_iota(jnp.int32, (1, NCELL), 1)
        p = r // GRID
        q = r - p * GRID

        def conv(src, w0, b0, cin, cout):
            h = jnp.zeros((cout, NCELL), dtype=jnp.float32)
            for dd in range(9):
                di, dj = dd // 3 - 1, dd % 3 - 1
                off = di * GRID + dj
                valid = ((p + di >= 0) & (p + di < GRID) &
                         (q + dj >= 0) & (q + dj < GRID))
                shifted = jnp.roll(src, -off, axis=1) if off else src
                shifted = jnp.where(valid, shifted, 0.0)
                wdd = wp_ref[w0 + dd * cout:w0 + (dd + 1) * cout, :cin]
                h = h + jnp.dot(wdd, shifted,
                                preferred_element_type=jnp.float32)
            bias = jnp.transpose(wp_ref[b0:b0 + 1, :cout])
            return jax.nn.relu(h + bias)

        h1 = conv(gt, _W1, _B1, F, C1)
        h2 = conv(h1, _W2, _B2, C1, C2)

        rr = jax.lax.broadcasted_iota(jnp.int32, (NCELL, PG * PG), 0)
        ss = jax.lax.broadcasted_iota(jnp.int32, (NCELL, PG * PG), 1)
        pm = ((rr // (GRID * POOL) == ss // PG) &
              ((rr % GRID) // POOL == ss % PG))
        pmat = pm.astype(jnp.float32) * (1.0 / (POOL * POOL))
        pooled = jnp.dot(h2, pmat, preferred_element_type=jnp.float32)

        emb = jnp.concatenate([pooled[c:c + 1, :] for c in range(C2)], axis=1)

        def head(hidx, nout):
            o = _HEADS + hidx * _HSZ
            w1 = wp_ref[o:o + EMB, :]
            b1 = wp_ref[o + EMB:o + EMB + 1, :]
            w2 = wp_ref[o + EMB + 1:o + EMB + 129, :C2]
            b2 = wp_ref[o + EMB + 129:o + EMB + 130, :C2]
            w3 = wp_ref[o + EMB + 130:o + EMB + 162, :nout]
            b3 = wp_ref[o + EMB + 162:o + EMB + 163, :nout]
            h = jax.nn.relu(jnp.dot(emb, w1,
                                    preferred_element_type=jnp.float32) + b1)
            h = jax.nn.relu(jnp.dot(h, w2,
                                    preferred_element_type=jnp.float32) + b2)
            return jnp.dot(h, w3, preferred_element_type=jnp.float32) + b3

        clf_ref[...] = head(0, 8)
        reg_ref[...] = head(1, 6)
        sin_o = jnp.tanh(head(2, 1))
        cos_o = jnp.tanh(head(3, 1))
        cyc_ref[...] = jnp.concatenate([sin_o, cos_o], axis=1)


def _padw(a, rows=None, cols=128):
    if a.ndim == 1:
        a = a.reshape(1, -1)
    r, c = a.shape
    return jnp.pad(a, ((0, (rows or r) - r), (0, cols - c)))


def kernel(ventana, params):
    nwin, npts, _ = ventana.shape
    num_chunks = nwin * npts // CHUNK
    x = ventana[:, :, 0].reshape(num_chunks, 1, CHUNK)
    y = ventana[:, :, 1].reshape(num_chunks, 1, CHUNK)
    v = ventana[:, :, 3].reshape(num_chunks, 1, CHUNK)

    w1taps = jnp.transpose(params["conv1"][0], (2, 3, 0, 1)).reshape(9 * C1, F)
    w2taps = jnp.transpose(params["conv2"][0], (2, 3, 0, 1)).reshape(9 * C2, C1)

    pieces = [
        _padw(params["enc"][0].reshape(1, F)),
        _padw(params["enc"][1]),
        _padw(w1taps),
        _padw(params["conv1"][1]),
        _padw(w2taps),
        _padw(params["conv2"][1]),
    ]
    for name in ("clf", "reg", "sin", "cos"):
        for w, b in params[name]:
            pieces.append(_padw(w))
            pieces.append(_padw(b))
    wpack = jnp.concatenate(pieces, axis=0)
    assert wpack.shape == (_ROWS, 128), wpack.shape

    chunk_spec = pl.BlockSpec((1, 1, CHUNK), lambda i: (i, 0, 0))

    logits, reg_out, cyc_out = pl.pallas_call(
        functools.partial(_fused_kernel, num_chunks=num_chunks),
        grid=(num_chunks,),
        in_specs=[chunk_spec, chunk_spec, chunk_spec,
                  pl.BlockSpec(wpack.shape, lambda i: (0, 0))],
        out_specs=(pl.BlockSpec((1, 8), lambda i: (0, 0)),
                   pl.BlockSpec((1, 6), lambda i: (0, 0)),
                   pl.BlockSpec((1, 2), lambda i: (0, 0))),
        out_shape=(jax.ShapeDtypeStruct((1, 8), jnp.float32),
                   jax.ShapeDtypeStruct((1, 6), jnp.float32),
                   jax.ShapeDtypeStruct((1, 2), jnp.float32)),
        scratch_shapes=[pltpu.VMEM((F, NPAD), jnp.float32)],
    )(x, y, v, wpack)

    return (logits, reg_out, cyc_out)


# speedup vs baseline: 1.9104x; 1.9104x over previous
import functools

import jax
import jax.numpy as jnp
from jax.experimental import pallas as pl
from jax.experimental.pallas import tpu as pltpu

GRID = 24
RES = 0.25
HALF = GRID * RES / 2.0
NCELL = GRID * GRID
NPAD = 640
CHUNK = 4096
F = 128
C1 = 64
C2 = 32
POOL = 4
PG = GRID // POOL
EMB = C2 * PG * PG


def _fused_kernel(x_ref, y_ref, v_ref, encw_ref, encb_ref,
                  w1_ref, b1_ref, w2_ref, b2_ref,
                  hw1_ref, hb1_ref, hw2_ref, hb2_ref, hw3_ref, hb3_ref,
                  clf_ref, reg_ref, cyc_ref, acc_ref,
                  *, num_chunks):
    i = pl.program_id(0)

    @pl.when(i == 0)
    def _init():
        acc_ref[...] = jnp.zeros_like(acc_ref)

    x = x_ref[0]
    y = y_ref[0]
    v = v_ref[0]
    cx = ((x + HALF) / RES).astype(jnp.int32)
    cy = ((y + HALF) / RES).astype(jnp.int32)
    mask = (cx >= 0) & (cx < GRID) & (cy >= 0) & (cy < GRID)
    idx = jnp.where(mask, cx * GRID + cy, NCELL)

    featT = jax.nn.relu(encw_ref[...] * v + encb_ref[...])

    cell_ids = jax.lax.broadcasted_iota(jnp.int32, (NPAD, CHUNK), 0)
    onehotT = (cell_ids == idx).astype(jnp.float32)

    acc_ref[...] += jax.lax.dot_general(
        featT, onehotT, (((1,), (1,)), ((), ())),
        preferred_element_type=jnp.float32)

    @pl.when(i == num_chunks - 1)
    def _tail():
        gt = acc_ref[:, :NCELL]

        r = jax.lax.broadcasted_iota(jnp.int32, (1, NCELL), 1)
        p = r // GRID
        q = r - p * GRID

        def conv(src, w_ref, b_ref, cout):
            h = jnp.zeros((cout, NCELL), dtype=jnp.float32)
            for dd in range(9):
                di, dj = dd // 3 - 1, dd % 3 - 1
                off = di * GRID + dj
                valid = ((p + di >= 0) & (p + di < GRID) &
                         (q + dj >= 0) & (q + dj < GRID))
                shifted = jnp.roll(src, -off, axis=1) if off else src
                shifted = jnp.where(valid, shifted, 0.0)
                wdd = w_ref[dd * cout:(dd + 1) * cout, :]
                h = h + jnp.dot(wdd, shifted,
                                preferred_element_type=jnp.float32)
            return jax.nn.relu(h + b_ref[...])

        h1 = conv(gt, w1_ref, b1_ref, C1)
        h2 = conv(h1, w2_ref, b2_ref, C2)

        rr = jax.lax.broadcasted_iota(jnp.int32, (NCELL, PG * PG), 0)
        ss = jax.lax.broadcasted_iota(jnp.int32, (NCELL, PG * PG), 1)
        pm = ((rr // (GRID * POOL) == ss // PG) &
              ((rr % GRID) // POOL == ss % PG))
        pmat = pm.astype(jnp.float32) * (1.0 / (POOL * POOL))
        pooled = jnp.dot(h2, pmat, preferred_element_type=jnp.float32)

        emb = jnp.concatenate([pooled[c:c + 1, :] for c in range(C2)], axis=1)

        def head(h_idx, nout):
            h = jax.nn.relu(jnp.dot(emb, hw1_ref[h_idx],
                                    preferred_element_type=jnp.float32)
                            + hb1_ref[h_idx:h_idx + 1, :])
            h = jax.nn.relu(jnp.dot(h, hw2_ref[h_idx],
                                    preferred_element_type=jnp.float32)
                            + hb2_ref[h_idx:h_idx + 1, :])
            return (jnp.dot(h, hw3_ref[h_idx][:, :nout],
                            preferred_element_type=jnp.float32)
                    + hb3_ref[h_idx:h_idx + 1, :nout])

        clf_ref[...] = head(0, 8)
        reg_ref[...] = head(1, 6)
        sin_o = jnp.tanh(head(2, 1))
        cos_o = jnp.tanh(head(3, 1))
        cyc_ref[...] = jnp.concatenate([sin_o, cos_o], axis=1)


def kernel(ventana, params):
    nwin, npts, _ = ventana.shape
    num_chunks = nwin * npts // CHUNK
    x = ventana[:, :, 0].reshape(num_chunks, 1, CHUNK)
    y = ventana[:, :, 1].reshape(num_chunks, 1, CHUNK)
    v = ventana[:, :, 3].reshape(num_chunks, 1, CHUNK)

    encw = params["enc"][0].reshape(1, F).T
    encb = params["enc"][1].reshape(1, F).T
    w1 = jnp.transpose(params["conv1"][0], (2, 3, 0, 1)).reshape(9 * C1, F)
    b1 = params["conv1"][1].reshape(C1, 1)
    w2 = jnp.transpose(params["conv2"][0], (2, 3, 0, 1)).reshape(9 * C2, C1)
    b2 = params["conv2"][1].reshape(C2, 1)

    hs = [params[n] for n in ("clf", "reg", "sin", "cos")]
    hw1 = jnp.stack([h[0][0] for h in hs])
    hb1 = jnp.stack([h[0][1] for h in hs])
    hw2 = jnp.stack([h[1][0] for h in hs])
    hb2 = jnp.stack([h[1][1] for h in hs])
    hw3 = jnp.stack([jnp.pad(h[2][0], ((0, 0), (0, 8 - h[2][0].shape[1])))
                     for h in hs])
    hb3 = jnp.stack([jnp.pad(h[2][1], (0, 8 - h[2][1].shape[0]))
                     for h in hs])
    head_args = (hw1, hb1, hw2, hb2, hw3, hb3)

    chunk_spec = pl.BlockSpec((1, 1, CHUNK), lambda i: (i, 0, 0))
    full = lambda a: pl.BlockSpec(a.shape, lambda i: (0,) * a.ndim)
    weights = (encw, encb, w1, b1, w2, b2) + head_args

    logits, reg_out, cyc_out = pl.pallas_call(
        functools.partial(_fused_kernel, num_chunks=num_chunks),
        grid=(num_chunks,),
        in_specs=[chunk_spec, chunk_spec, chunk_spec] +
                 [full(a) for a in weights],
        out_specs=(pl.BlockSpec((1, 8), lambda i: (0, 0)),
                   pl.BlockSpec((1, 6), lambda i: (0, 0)),
                   pl.BlockSpec((1, 2), lambda i: (0, 0))),
        out_shape=(jax.ShapeDtypeStruct((1, 8), jnp.float32),
                   jax.ShapeDtypeStruct((1, 6), jnp.float32),
                   jax.ShapeDtypeStruct((1, 2), jnp.float32)),
        scratch_shapes=[pltpu.VMEM((F, NPAD), jnp.float32)],
    )(x, y, v, *weights)

    return (logits, reg_out, cyc_out)


# speedup vs baseline: 1.9880x; 1.0406x over previous
import functools

import jax
import jax.numpy as jnp
from jax.experimental import pallas as pl
from jax.experimental.pallas import tpu as pltpu

GRID = 24
RES = 0.25
HALF = GRID * RES / 2.0
NCELL = GRID * GRID
NPAD = 640
CHUNK = 32768
F = 128
C1 = 64
C2 = 32
POOL = 4
PG = GRID // POOL
EMB = C2 * PG * PG


def _fused_kernel(x_ref, y_ref, v_ref, encw_ref, encb_ref,
                  w1_ref, b1_ref, w2_ref, b2_ref,
                  hw1_ref, hb1_ref, hw2_ref, hb2_ref, hw3_ref, hb3_ref,
                  clf_ref, reg_ref, cyc_ref, acc_ref,
                  *, num_chunks):
    i = pl.program_id(0)

    @pl.when(i == 0)
    def _init():
        acc_ref[...] = jnp.zeros_like(acc_ref)

    x = x_ref[0]
    y = y_ref[0]
    v = v_ref[0]
    cx = ((x + HALF) / RES).astype(jnp.int32)
    cy = ((y + HALF) / RES).astype(jnp.int32)
    mask = (cx >= 0) & (cx < GRID) & (cy >= 0) & (cy < GRID)
    idx = jnp.where(mask, cx * GRID + cy, NCELL)

    featT = jax.nn.relu(encw_ref[...] * v + encb_ref[...])

    cell_ids = jax.lax.broadcasted_iota(jnp.int32, (NPAD, CHUNK), 0)
    onehotT = (cell_ids == idx).astype(jnp.float32)

    acc_ref[...] += jax.lax.dot_general(
        featT, onehotT, (((1,), (1,)), ((), ())),
        preferred_element_type=jnp.float32)

    @pl.when(i == num_chunks - 1)
    def _tail():
        gt = acc_ref[:, :NCELL]

        r = jax.lax.broadcasted_iota(jnp.int32, (1, NCELL), 1)
        p = r // GRID
        q = r - p * GRID

        def conv(src, w_ref, b_ref, cout):
            h = jnp.zeros((cout, NCELL), dtype=jnp.float32)
            for dd in range(9):
                di, dj = dd // 3 - 1, dd % 3 - 1
                off = di * GRID + dj
                valid = ((p + di >= 0) & (p + di < GRID) &
                         (q + dj >= 0) & (q + dj < GRID))
                shifted = jnp.roll(src, -off, axis=1) if off else src
                shifted = jnp.where(valid, shifted, 0.0)
                wdd = w_ref[dd * cout:(dd + 1) * cout, :]
                h = h + jnp.dot(wdd, shifted,
                                preferred_element_type=jnp.float32)
            return jax.nn.relu(h + b_ref[...])

        h1 = conv(gt, w1_ref, b1_ref, C1)
        h2 = conv(h1, w2_ref, b2_ref, C2)

        rr = jax.lax.broadcasted_iota(jnp.int32, (NCELL, PG * PG), 0)
        ss = jax.lax.broadcasted_iota(jnp.int32, (NCELL, PG * PG), 1)
        pm = ((rr // (GRID * POOL) == ss // PG) &
              ((rr % GRID) // POOL == ss % PG))
        pmat = pm.astype(jnp.float32) * (1.0 / (POOL * POOL))
        pooled = jnp.dot(h2, pmat, preferred_element_type=jnp.float32)

        emb = jnp.concatenate([pooled[c:c + 1, :] for c in range(C2)], axis=1)

        def head(h_idx, nout):
            h = jax.nn.relu(jnp.dot(emb, hw1_ref[h_idx],
                                    preferred_element_type=jnp.float32)
                            + hb1_ref[h_idx:h_idx + 1, :])
            h = jax.nn.relu(jnp.dot(h, hw2_ref[h_idx],
                                    preferred_element_type=jnp.float32)
                            + hb2_ref[h_idx:h_idx + 1, :])
            return (jnp.dot(h, hw3_ref[h_idx][:, :nout],
                            preferred_element_type=jnp.float32)
                    + hb3_ref[h_idx:h_idx + 1, :nout])

        clf_ref[...] = head(0, 8)
        reg_ref[...] = head(1, 6)
        sin_o = jnp.tanh(head(2, 1))
        cos_o = jnp.tanh(head(3, 1))
        cyc_ref[...] = jnp.concatenate([sin_o, cos_o], axis=1)


def kernel(ventana, params):
    nwin, npts, _ = ventana.shape
    num_chunks = nwin * npts // CHUNK
    x = ventana[:, :, 0].reshape(num_chunks, 1, CHUNK)
    y = ventana[:, :, 1].reshape(num_chunks, 1, CHUNK)
    v = ventana[:, :, 3].reshape(num_chunks, 1, CHUNK)

    encw = params["enc"][0].reshape(1, F).T
    encb = params["enc"][1].reshape(1, F).T
    w1 = jnp.transpose(params["conv1"][0], (2, 3, 0, 1)).reshape(9 * C1, F)
    b1 = params["conv1"][1].reshape(C1, 1)
    w2 = jnp.transpose(params["conv2"][0], (2, 3, 0, 1)).reshape(9 * C2, C1)
    b2 = params["conv2"][1].reshape(C2, 1)

    hs = [params[n] for n in ("clf", "reg", "sin", "cos")]
    hw1 = jnp.stack([h[0][0] for h in hs])
    hb1 = jnp.stack([h[0][1] for h in hs])
    hw2 = jnp.stack([h[1][0] for h in hs])
    hb2 = jnp.stack([h[1][1] for h in hs])
    hw3 = jnp.stack([jnp.pad(h[2][0], ((0, 0), (0, 8 - h[2][0].shape[1])))
                     for h in hs])
    hb3 = jnp.stack([jnp.pad(h[2][1], (0, 8 - h[2][1].shape[0]))
                     for h in hs])
    head_args = (hw1, hb1, hw2, hb2, hw3, hb3)

    chunk_spec = pl.BlockSpec((1, 1, CHUNK), lambda i: (i, 0, 0))
    full = lambda a: pl.BlockSpec(a.shape, lambda i: (0,) * a.ndim)
    weights = (encw, encb, w1, b1, w2, b2) + head_args

    logits, reg_out, cyc_out = pl.pallas_call(
        functools.partial(_fused_kernel, num_chunks=num_chunks),
        grid=(num_chunks,),
        in_specs=[chunk_spec, chunk_spec, chunk_spec] +
                 [full(a) for a in weights],
        out_specs=(pl.BlockSpec((1, 8), lambda i: (0, 0)),
                   pl.BlockSpec((1, 6), lambda i: (0, 0)),
                   pl.BlockSpec((1, 2), lambda i: (0, 0))),
        out_shape=(jax.ShapeDtypeStruct((1, 8), jnp.float32),
                   jax.ShapeDtypeStruct((1, 6), jnp.float32),
                   jax.ShapeDtypeStruct((1, 2), jnp.float32)),
        scratch_shapes=[pltpu.VMEM((F, NPAD), jnp.float32)],
    )(x, y, v, *weights)

    return (logits, reg_out, cyc_out)


# speedup vs baseline: 2.2763x; 1.1450x over previous
import functools

import jax
import jax.numpy as jnp
from jax.experimental import pallas as pl
from jax.experimental.pallas import tpu as pltpu

GRID = 24
RES = 0.25
HALF = GRID * RES / 2.0
NCELL = GRID * GRID
NPAD = 640
CHUNK = 4096
F = 128
C1 = 64
C2 = 32
POOL = 4
PG = GRID // POOL
EMB = C2 * PG * PG
NW = 28


def _fused_kernel(x_ref, y_ref, v_ref, encw_ref, encb_ref, *refs,
                  num_chunks):
    hbm = refs[:NW]
    clf_ref, reg_ref, cyc_ref = refs[NW:NW + 3]
    acc_ref = refs[NW + 3]
    wvm = refs[NW + 4:NW + 4 + NW]
    sem = refs[NW + 4 + NW]

    i = pl.program_id(0)

    def copies():
        return [pltpu.make_async_copy(hbm[j], wvm[j], sem.at[j])
                for j in range(NW)]

    @pl.when(i == 0)
    def _init():
        acc_ref[...] = jnp.zeros_like(acc_ref)
        for c in copies():
            c.start()

    x = x_ref[0]
    y = y_ref[0]
    v = v_ref[0]
    cx = ((x + HALF) / RES).astype(jnp.int32)
    cy = ((y + HALF) / RES).astype(jnp.int32)
    mask = (cx >= 0) & (cx < GRID) & (cy >= 0) & (cy < GRID)
    idx = jnp.where(mask, cx * GRID + cy, NCELL)

    encw = jnp.transpose(encw_ref[...])
    encb = jnp.transpose(encb_ref[...])
    featT = jax.nn.relu(encw * v + encb)

    cell_ids = jax.lax.broadcasted_iota(jnp.int32, (NPAD, CHUNK), 0)
    onehotT = (cell_ids == idx).astype(jnp.float32)

    acc_ref[...] += jax.lax.dot_general(
        featT, onehotT, (((1,), (1,)), ((), ())),
        preferred_element_type=jnp.float32)

    @pl.when(i == num_chunks - 1)
    def _tail():
        for c in copies():
            c.wait()
        w1flat, b1, w2flat, b2 = (wvm[0], wvm[1], wvm[2], wvm[3])

        gt = acc_ref[:, :NCELL]

        r = jax.lax.broadcasted_iota(jnp.int32, (1, NCELL), 1)
        p = r // GRID
        q = r - p * GRID

        def conv(src, wflat_ref, b_ref, cin, cout):
            kk = jax.lax.broadcasted_iota(jnp.int32, (cin * 9, cin), 0)
            cc = jax.lax.broadcasted_iota(jnp.int32, (cin * 9, cin), 1)
            h = jnp.zeros((cout, NCELL), dtype=jnp.float32)
            for dd in range(9):
                di, dj = dd // 3 - 1, dd % 3 - 1
                off = di * GRID + dj
                valid = ((p + di >= 0) & (p + di < GRID) &
                         (q + dj >= 0) & (q + dj < GRID))
                shifted = jnp.roll(src, -off, axis=1) if off else src
                shifted = jnp.where(valid, shifted, 0.0)
                sel = (kk == cc * 9 + dd).astype(jnp.float32)
                wdd = jnp.dot(wflat_ref[...], sel,
                              preferred_element_type=jnp.float32)
                h = h + jnp.dot(wdd, shifted,
                                preferred_element_type=jnp.float32)
            bias = jnp.transpose(b_ref[...])
            return jax.nn.relu(h + bias)

        h1 = conv(gt, w1flat, b1, F, C1)
        h2 = conv(h1, w2flat, b2, C1, C2)

        rr = jax.lax.broadcasted_iota(jnp.int32, (NCELL, PG * PG), 0)
        ss = jax.lax.broadcasted_iota(jnp.int32, (NCELL, PG * PG), 1)
        pm = ((rr // (GRID * POOL) == ss // PG) &
              ((rr % GRID) // POOL == ss % PG))
        pmat = pm.astype(jnp.float32) * (1.0 / (POOL * POOL))
        pooled = jnp.dot(h2, pmat, preferred_element_type=jnp.float32)

        emb = jnp.concatenate([pooled[c:c + 1, :] for c in range(C2)], axis=1)

        def head(h_idx, nout):
            o = 4 + h_idx * 6
            h = jax.nn.relu(jnp.dot(emb, wvm[o][...],
                                    preferred_element_type=jnp.float32)
                            + wvm[o + 1][...])
            h = jax.nn.relu(jnp.dot(h, wvm[o + 2][...],
                                    preferred_element_type=jnp.float32)
                            + wvm[o + 3][...])
            return (jnp.dot(h, wvm[o + 4][...],
                            preferred_element_type=jnp.float32)
                    + wvm[o + 5][...])

        clf_ref[...] = head(0, 8)
        reg_ref[...] = head(1, 6)
        sin_o = jnp.tanh(head(2, 1))
        cos_o = jnp.tanh(head(3, 1))
        cyc_ref[...] = jnp.concatenate([sin_o, cos_o], axis=1)


def kernel(ventana, params):
    nwin, npts, _ = ventana.shape
    num_chunks = nwin * npts // CHUNK
    x = ventana[:, :, 0].reshape(num_chunks, 1, CHUNK)
    y = ventana[:, :, 1].reshape(num_chunks, 1, CHUNK)
    v = ventana[:, :, 3].reshape(num_chunks, 1, CHUNK)

    encw = params["enc"][0]
    encb = params["enc"][1].reshape(1, F)

    weights = [params["conv1"][0].reshape(C1, F * 9),
               params["conv1"][1].reshape(1, C1),
               params["conv2"][0].reshape(C2, C1 * 9),
               params["conv2"][1].reshape(1, C2)]
    for name in ("clf", "reg", "sin", "cos"):
        for w, b in params[name]:
            weights.append(w)
            weights.append(b.reshape(1, -1))
    assert len(weights) == NW

    chunk_spec = pl.BlockSpec((1, 1, CHUNK), lambda i: (i, 0, 0))
    full = lambda a: pl.BlockSpec(a.shape, lambda i: (0,) * a.ndim)
    any_spec = pl.BlockSpec(memory_space=pltpu.MemorySpace.HBM)

    logits, reg_out, cyc_out = pl.pallas_call(
        functools.partial(_fused_kernel, num_chunks=num_chunks),
        grid=(num_chunks,),
        in_specs=[chunk_spec, chunk_spec, chunk_spec,
                  full(encw), full(encb)] + [any_spec] * NW,
        out_specs=(pl.BlockSpec((1, 8), lambda i: (0, 0)),
                   pl.BlockSpec((1, 6), lambda i: (0, 0)),
                   pl.BlockSpec((1, 2), lambda i: (0, 0))),
        out_shape=(jax.ShapeDtypeStruct((1, 8), jnp.float32),
                   jax.ShapeDtypeStruct((1, 6), jnp.float32),
                   jax.ShapeDtypeStruct((1, 2), jnp.float32)),
        scratch_shapes=([pltpu.VMEM((F, NPAD), jnp.float32)] +
                        [pltpu.VMEM(w.shape, jnp.float32) for w in weights] +
                        [pltpu.SemaphoreType.DMA((NW,))]),
    )(x, y, v, encw, encb, *weights)

    return (logits, reg_out, cyc_out)
